# baseline (speedup 1.0000x reference)
import functools

import jax
import jax.numpy as jnp
from jax.experimental import pallas as pl
from jax.experimental.pallas import tpu as pltpu
from jax.experimental.pallas import tpu_sc as plsc

B, T, D = 2, 2048, 768
H = 12
DH = D // H
E = 8
TOPK = 2
DFF = 2048
NTOK = B * T
NCH = 2
N = NTOK // NCH
C = 1024
DROP = E * C
RB = 512
F32 = jnp.float32
BF16 = jnp.bfloat16


def _ln(x, g, b):
    m = jnp.mean(x, axis=1, keepdims=True)
    v = jnp.mean((x - m) ** 2, axis=1, keepdims=True)
    return (x - m) / jnp.sqrt(v + 1e-5) * g + b


def _qkv_body(x_ref, g_ref, b_ref, w_ref, bias_ref, o_ref):
    xn = _ln(x_ref[...], g_ref[...], b_ref[...])
    o_ref[...] = (jnp.dot(xn.astype(BF16), w_ref[...], preferred_element_type=F32)
                  + bias_ref[...]).astype(BF16)


def _qkv(xf, g1, b1, Wqkv, bqkv):
    return pl.pallas_call(
        _qkv_body,
        grid=(NTOK // RB,),
        in_specs=[
            pl.BlockSpec((RB, D), lambda i: (i, 0)),
            pl.BlockSpec((1, D), lambda i: (0, 0)),
            pl.BlockSpec((1, D), lambda i: (0, 0)),
            pl.BlockSpec((D, 3 * D), lambda i: (0, 0)),
            pl.BlockSpec((1, 3 * D), lambda i: (0, 0)),
        ],
        out_specs=pl.BlockSpec((RB, 3 * D), lambda i: (i, 0)),
        out_shape=jax.ShapeDtypeStruct((NTOK, 3 * D), BF16),
    )(xf, g1.reshape(1, D), b1.reshape(1, D), Wqkv.astype(BF16),
      bqkv.reshape(1, 3 * D))


QB = 1024


def _attn_body(q_ref, k_ref, v_ref, o_ref):
    q2 = q_ref[...]
    k2 = k_ref[...]
    v2 = v_ref[...]
    outs = []
    for sub in range(2):
        q = q2[:, sub * DH:(sub + 1) * DH]
        k = k2[:, sub * DH:(sub + 1) * DH]
        v = v2[:, sub * DH:(sub + 1) * DH]
        s = jax.lax.dot_general(q, k, (((1,), (1,)), ((), ())),
                                preferred_element_type=F32) * (1.0 / 8.0)
        p = jnp.exp(s)
        l = jnp.sum(p, axis=1, keepdims=True)
        o = jnp.dot(p.astype(BF16), v, preferred_element_type=F32)
        outs.append((o / l).astype(BF16))
    o_ref[...] = jnp.concatenate(outs, axis=1)


def _attn(qkv):
    HP = H // 2
    return pl.pallas_call(
        _attn_body,
        grid=(B, HP, T // QB),
        in_specs=[
            pl.BlockSpec((QB, 2 * DH), lambda b, h, i: (b * (T // QB) + i, h)),
            pl.BlockSpec((T, 2 * DH), lambda b, h, i: (b, HP + h)),
            pl.BlockSpec((T, 2 * DH), lambda b, h, i: (b, 2 * HP + h)),
        ],
        out_specs=pl.BlockSpec((QB, 2 * DH), lambda b, h, i: (b * (T // QB) + i, h)),
        out_shape=jax.ShapeDtypeStruct((NTOK, D), BF16),
    )(qkv, qkv, qkv)


def _proj_body(a_ref, wo_ref, bo_ref, res_ref, g_ref, b_ref, wg_ref,
               xo_ref, ln_ref, lg_ref):
    o = jnp.dot(a_ref[...], wo_ref[...], preferred_element_type=F32) + bo_ref[...]
    xo = o + res_ref[...]
    xo_ref[...] = xo
    ln = _ln(xo, g_ref[...], b_ref[...])
    li = jax.lax.bitcast_convert_type(ln[:, :D // 2], jnp.int32)
    hi = jax.lax.bitcast_convert_type(ln[:, D // 2:], jnp.int32)
    lo_b = jax.lax.shift_right_logical(li + 0x8000, 16)
    hi_b = (hi + 0x8000) & jnp.int32(-65536)
    ln_ref[...] = lo_b | hi_b
    lg_ref[...] = jnp.dot(ln, wg_ref[...], preferred_element_type=F32)


def _proj(attn, Wo, bo, xf, g2, b2, Wgp):
    return pl.pallas_call(
        _proj_body,
        grid=(NTOK // RB,),
        in_specs=[
            pl.BlockSpec((RB, D), lambda i: (i, 0)),
            pl.BlockSpec((D, D), lambda i: (0, 0)),
            pl.BlockSpec((1, D), lambda i: (0, 0)),
            pl.BlockSpec((RB, D), lambda i: (i, 0)),
            pl.BlockSpec((1, D), lambda i: (0, 0)),
            pl.BlockSpec((1, D), lambda i: (0, 0)),
            pl.BlockSpec((D, 128), lambda i: (0, 0)),
        ],
        out_specs=[
            pl.BlockSpec((RB, D), lambda i: (i, 0)),
            pl.BlockSpec((RB, D // 2), lambda i: (i, 0)),
            pl.BlockSpec((RB, 128), lambda i: (i, 0)),
        ],
        out_shape=[
            jax.ShapeDtypeStruct((NTOK, D), F32),
            jax.ShapeDtypeStruct((NTOK, D // 2), jnp.int32),
            jax.ShapeDtypeStruct((NTOK, 128), F32),
        ],
    )(attn, Wo.astype(BF16), bo.reshape(1, D), xf, g2.reshape(1, D),
      b2.reshape(1, D), Wgp)


def _route_body(lg_ref, d0_ref, d1_ref, g0_ref, g1_ref, w0_ref, w1_ref, cnt_ref):
    lg = lg_ref[0]
    col = jax.lax.broadcasted_iota(jnp.int32, (N, 128), 1)
    valid = col < E
    lgm = jnp.where(valid, lg, -1e30)
    mx = jnp.max(lgm, axis=1, keepdims=True)
    p = jnp.where(valid, jnp.exp(lgm - mx), 0.0)
    probs = p / jnp.sum(p, axis=1, keepdims=True)
    w0 = jnp.max(probs, axis=1, keepdims=True)
    i0 = jnp.min(jnp.where((probs == w0) & valid, col, 128), axis=1, keepdims=True)
    oh0 = col == i0
    probs1 = jnp.where(oh0 | ~valid, -1.0, probs)
    w1 = jnp.max(probs1, axis=1, keepdims=True)
    i1 = jnp.min(jnp.where(probs1 == w1, col, 128), axis=1, keepdims=True)
    oh1 = col == i1
    wsum = w0 + w1
    w0n = w0 / wsum
    w1n = w1 / wsum
    ohs = oh0.astype(F32) + oh1.astype(F32)
    y = ohs
    sh = 1
    while sh < N:
        y = y + jnp.concatenate([jnp.zeros((sh, 128), F32), y[:N - sh]], axis=0)
        sh *= 2
    cnt_ref[0] = y[N - 1:N]
    Sx = y - ohs
    oh0f = oh0.astype(F32)
    r0 = jnp.sum(Sx * oh0f, axis=1, keepdims=True).astype(jnp.int32)
    r1 = jnp.sum((Sx + oh0f) * oh1.astype(F32), axis=1, keepdims=True).astype(jnp.int32)
    keep0 = r0 < C
    keep1 = r1 < C
    d0_ref[0] = jnp.where(keep0, i0 * C + r0, DROP)
    d1_ref[0] = jnp.where(keep1, i1 * C + r1, DROP)
    g0_ref[0] = i0 * C + jnp.minimum(r0, C - 1)
    g1_ref[0] = i1 * C + jnp.minimum(r1, C - 1)
    w0_ref[0] = jnp.where(keep0, w0n, 0.0)
    w1_ref[0] = jnp.where(keep1, w1n, 0.0)


def _route(logits):
    ospec = pl.BlockSpec((1, N, 1), lambda c: (c, 0, 0))
    oshape_i = jax.ShapeDtypeStruct((NCH, N, 1), jnp.int32)
    oshape_f = jax.ShapeDtypeStruct((NCH, N, 1), F32)
    return pl.pallas_call(
        _route_body,
        grid=(NCH,),
        in_specs=[pl.BlockSpec((1, N, 128), lambda c: (c, 0, 0))],
        out_specs=[ospec] * 6 + [pl.BlockSpec((1, 1, 128), lambda c: (c, 0, 0))],
        out_shape=[oshape_i, oshape_i, oshape_i, oshape_i, oshape_f, oshape_f,
                   jax.ShapeDtypeStruct((NCH, 1, 128), F32)],
    )(logits.reshape(NCH, N, 128))


NW = 32
SCB = 128
CPAD = E * C + 8


def _sc_mesh():
    return plsc.VectorSubcoreMesh(core_axis_name="c", subcore_axis_name="s")


def _sc_dispatch(table, srcidx, dstidx):
    nidx = srcidx.shape[0]
    per_w = nidx // NW

    @functools.partial(
        pl.kernel, mesh=_sc_mesh(),
        out_type=jax.ShapeDtypeStruct((NCH * CPAD, D // 2), jnp.int32),
        scratch_types=[
            pltpu.VMEM((SCB,), jnp.int32),
            pltpu.VMEM((SCB,), jnp.int32),
            pltpu.VMEM((SCB, D // 2), jnp.int32),
            pltpu.SemaphoreType.DMA,
            pltpu.SemaphoreType.DMA,
        ],
    )
    def k(table_hbm, src_hbm, dst_hbm, out_hbm, src_v, dst_v, rows_v, s1, s2):
        wid = jax.lax.axis_index("s") * 2 + jax.lax.axis_index("c")

        def body(j, carry):
            base = pl.multiple_of(wid * per_w + j * SCB, SCB)
            pltpu.sync_copy(src_hbm.at[pl.ds(base, SCB)], src_v)
            pltpu.async_copy(table_hbm.at[src_v], rows_v, s1).wait()
            pltpu.sync_copy(dst_hbm.at[pl.ds(base, SCB)], dst_v)
            pltpu.async_copy(rows_v, out_hbm.at[dst_v], s2).wait()
            return carry

        jax.lax.fori_loop(0, per_w // SCB, body, 0)

    return k(table, srcidx, dstidx)


def _sc_gather(table, idx):
    nidx = idx.shape[0]
    per_w = nidx // NW

    @functools.partial(
        pl.kernel, mesh=_sc_mesh(),
        out_type=jax.ShapeDtypeStruct((nidx, D), F32),
        scratch_types=[
            pltpu.VMEM((SCB,), jnp.int32),
            pltpu.VMEM((SCB, D), F32),
            pltpu.SemaphoreType.DMA,
        ],
    )
    def k(table_hbm, idx_hbm, out_hbm, idx_v, rows_v, s1):
        wid = jax.lax.axis_index("s") * 2 + jax.lax.axis_index("c")

        def body(j, carry):
            base = pl.multiple_of(wid * per_w + j * SCB, SCB)
            pltpu.sync_copy(idx_hbm.at[pl.ds(base, SCB)], idx_v)
            pltpu.async_copy(table_hbm.at[idx_v], rows_v, s1).wait()
            pltpu.sync_copy(rows_v, out_hbm.at[pl.ds(base, SCB)])
            return carry

        jax.lax.fori_loop(0, per_w // SCB, body, 0)

    return k(table, idx)


TB = 512


def _wsum_body(w0_ref, w1_ref, x_ref, a_ref, b_ref, o_ref):
    o_ref[...] = (x_ref[...] + w0_ref[0] * a_ref[...] + w1_ref[0] * b_ref[...])


def _wsum(w0, w1, xout, gath):
    nt = N // TB
    return pl.pallas_call(
        _wsum_body,
        grid=(NCH, nt),
        in_specs=[
            pl.BlockSpec((1, TB, 1), lambda c, t: (c, t, 0)),
            pl.BlockSpec((1, TB, 1), lambda c, t: (c, t, 0)),
            pl.BlockSpec((TB, D), lambda c, t: (c * (N // TB) + t, 0)),
            pl.BlockSpec((TB, D), lambda c, t: (c * (N // TB) + t, 0)),
            pl.BlockSpec((TB, D), lambda c, t: (NTOK // TB + c * (N // TB) + t, 0)),
        ],
        out_specs=pl.BlockSpec((TB, D), lambda c, t: (c * (N // TB) + t, 0)),
        out_shape=jax.ShapeDtypeStruct((NTOK, D), F32),
    )(w0, w1, xout, gath, gath)


CB = 128


def _ffn_body(cnt_ref, buf_ref, w1_ref, b1_ref, w2_ref, b2_ref, o_ref, w1s, w2s):
    e = pl.program_id(0)
    c = pl.program_id(1)
    rb = pl.program_id(2)

    @pl.when((c == 0) & (rb == 0))
    def _():
        w1s[...] = w1_ref[0].astype(BF16)
        w2s[...] = w2_ref[0].astype(BF16)

    @pl.when(rb * CB < cnt_ref[c * E + e])
    def _():
        w = buf_ref[0]
        lo = jax.lax.bitcast_convert_type(jax.lax.shift_left(w, 16), F32)
        hi = jax.lax.bitcast_convert_type(w & jnp.int32(-65536), F32)
        x = jnp.concatenate([lo, hi], axis=1).astype(BF16)
        h = jnp.dot(x, w1s[...], preferred_element_type=F32) + b1_ref[0]
        h = jax.nn.gelu(h)
        o_ref[0] = jnp.dot(h.astype(BF16), w2s[...],
                           preferred_element_type=F32) + b2_ref[0]


def _ffn(cnt, buf, W1, b1r, W2, b2r):
    grid_spec = pltpu.PrefetchScalarGridSpec(
        num_scalar_prefetch=1,
        grid=(E, NCH, C // CB),
        in_specs=[
            pl.BlockSpec((1, CB, D // 2),
                         lambda e, c, r, s: (c, e * (C // CB) + r, 0)),
            pl.BlockSpec((1, D, DFF), lambda e, c, r, s: (e, 0, 0)),
            pl.BlockSpec((1, 1, DFF), lambda e, c, r, s: (e, 0, 0)),
            pl.BlockSpec((1, DFF, D), lambda e, c, r, s: (e, 0, 0)),
            pl.BlockSpec((1, 1, D), lambda e, c, r, s: (e, 0, 0)),
        ],
        out_specs=pl.BlockSpec((1, CB, D),
                               lambda e, c, r, s: (c, e * (C // CB) + r, 0)),
        scratch_shapes=[
            pltpu.VMEM((D, DFF), BF16),
            pltpu.VMEM((DFF, D), BF16),
        ],
    )
    return pl.pallas_call(
        _ffn_body,
        grid_spec=grid_spec,
        out_shape=jax.ShapeDtypeStruct((NCH, E * C, D), F32),
    )(cnt, buf, W1, b1r, W2, b2r)


def kernel(x, gamma1, beta1, Wqkv, bqkv, Wo, bo, gamma2, beta2, Wg, W1, b1, W2, b2):
    xf = x.reshape(NTOK, D)
    qkv = _qkv(xf, gamma1, beta1, Wqkv, bqkv)
    attn = _attn(qkv)
    Wgp = jnp.pad(Wg, ((0, 0), (0, 128 - E)))
    xout, lnf, logits = _proj(attn, Wo, bo, xf, gamma2, beta2, Wgp)
    d0, d1, g0, g1, w0, w1, cnte = _route(logits)
    cnt = cnte[:, 0, :E].astype(jnp.int32).reshape(NCH * E)
    coff = (jnp.arange(NCH, dtype=jnp.int32) * CPAD)[:, None, None]
    dst = jnp.concatenate([(d0 + coff).reshape(-1), (d1 + coff).reshape(-1)])
    src = jnp.concatenate([jnp.arange(NTOK, dtype=jnp.int32)] * 2)
    buf = _sc_dispatch(lnf, src, dst).reshape(NCH, CPAD, D // 2)
    eo = _ffn(cnt, buf, W1, b1.reshape(E, 1, DFF),
              W2, b2.reshape(E, 1, D))
    goff = (jnp.arange(NCH, dtype=jnp.int32) * (E * C))[:, None, None]
    gcat = jnp.concatenate([(g0 + goff).reshape(-1), (g1 + goff).reshape(-1)])
    gath = _sc_gather(eo.reshape(NCH * E * C, D), gcat)
    out = _wsum(w0, w1, xout, gath)
    return out.reshape(B, T, D)


# speedup vs baseline: 1.1029x; 1.1029x over previous
import functools

import jax
import jax.numpy as jnp
from jax.experimental import pallas as pl
from jax.experimental.pallas import tpu as pltpu
from jax.experimental.pallas import tpu_sc as plsc

B, T, D = 2, 2048, 768
H = 12
DH = D // H
E = 8
TOPK = 2
DFF = 2048
NTOK = B * T
NCH = 2
N = NTOK // NCH
C = 1024
DROP = E * C
RB = 512
F32 = jnp.float32
BF16 = jnp.bfloat16


def _ln(x, g, b):
    m = jnp.mean(x, axis=1, keepdims=True)
    v = jnp.mean((x - m) ** 2, axis=1, keepdims=True)
    return (x - m) / jnp.sqrt(v + 1e-5) * g + b


def _qkv_body(x_ref, g_ref, b_ref, w_ref, bias_ref, o_ref):
    xn = _ln(x_ref[...], g_ref[...], b_ref[...])
    o_ref[...] = (jnp.dot(xn.astype(BF16), w_ref[...], preferred_element_type=F32)
                  + bias_ref[...]).astype(BF16)


def _qkv(xf, g1, b1, Wqkv, bqkv):
    return pl.pallas_call(
        _qkv_body,
        grid=(NTOK // RB,),
        in_specs=[
            pl.BlockSpec((RB, D), lambda i: (i, 0)),
            pl.BlockSpec((1, D), lambda i: (0, 0)),
            pl.BlockSpec((1, D), lambda i: (0, 0)),
            pl.BlockSpec((D, 3 * D), lambda i: (0, 0)),
            pl.BlockSpec((1, 3 * D), lambda i: (0, 0)),
        ],
        out_specs=pl.BlockSpec((RB, 3 * D), lambda i: (i, 0)),
        out_shape=jax.ShapeDtypeStruct((NTOK, 3 * D), BF16),
    )(xf, g1.reshape(1, D), b1.reshape(1, D), Wqkv.astype(BF16),
      bqkv.reshape(1, 3 * D))


QB = 1024


def _attn_body(q_ref, k_ref, v_ref, o_ref):
    q2 = q_ref[...]
    k2 = k_ref[...]
    v2 = v_ref[...]
    outs = []
    for sub in range(2):
        q = q2[:, sub * DH:(sub + 1) * DH]
        k = k2[:, sub * DH:(sub + 1) * DH]
        v = v2[:, sub * DH:(sub + 1) * DH]
        s = jax.lax.dot_general(q, k, (((1,), (1,)), ((), ())),
                                preferred_element_type=F32) * (1.0 / 8.0)
        p = jnp.exp(s)
        l = jnp.sum(p, axis=1, keepdims=True)
        o = jnp.dot(p.astype(BF16), v, preferred_element_type=F32)
        outs.append((o / l).astype(BF16))
    o_ref[...] = jnp.concatenate(outs, axis=1)


def _attn(qkv):
    HP = H // 2
    return pl.pallas_call(
        _attn_body,
        grid=(B, HP, T // QB),
        in_specs=[
            pl.BlockSpec((QB, 2 * DH), lambda b, h, i: (b * (T // QB) + i, h)),
            pl.BlockSpec((T, 2 * DH), lambda b, h, i: (b, HP + h)),
            pl.BlockSpec((T, 2 * DH), lambda b, h, i: (b, 2 * HP + h)),
        ],
        out_specs=pl.BlockSpec((QB, 2 * DH), lambda b, h, i: (b * (T // QB) + i, h)),
        out_shape=jax.ShapeDtypeStruct((NTOK, D), BF16),
    )(qkv, qkv, qkv)


def _proj_body(a_ref, wo_ref, bo_ref, res_ref, g_ref, b_ref, wg_ref,
               xo_ref, ln_ref, lg_ref):
    o = jnp.dot(a_ref[...], wo_ref[...], preferred_element_type=F32) + bo_ref[...]
    xo = o + res_ref[...]
    xo_ref[...] = xo
    ln = _ln(xo, g_ref[...], b_ref[...])
    li = jax.lax.bitcast_convert_type(ln[:, :D // 2], jnp.int32)
    hi = jax.lax.bitcast_convert_type(ln[:, D // 2:], jnp.int32)
    lo_b = jax.lax.shift_right_logical(li + 0x8000, 16)
    hi_b = (hi + 0x8000) & jnp.int32(-65536)
    ln_ref[...] = lo_b | hi_b
    lg_ref[...] = jnp.dot(ln, wg_ref[...], preferred_element_type=F32)


def _proj(attn, Wo, bo, xf, g2, b2, Wgp):
    return pl.pallas_call(
        _proj_body,
        grid=(NTOK // RB,),
        in_specs=[
            pl.BlockSpec((RB, D), lambda i: (i, 0)),
            pl.BlockSpec((D, D), lambda i: (0, 0)),
            pl.BlockSpec((1, D), lambda i: (0, 0)),
            pl.BlockSpec((RB, D), lambda i: (i, 0)),
            pl.BlockSpec((1, D), lambda i: (0, 0)),
            pl.BlockSpec((1, D), lambda i: (0, 0)),
            pl.BlockSpec((D, 128), lambda i: (0, 0)),
        ],
        out_specs=[
            pl.BlockSpec((RB, D), lambda i: (i, 0)),
            pl.BlockSpec((RB, D // 2), lambda i: (i, 0)),
            pl.BlockSpec((RB, 128), lambda i: (i, 0)),
        ],
        out_shape=[
            jax.ShapeDtypeStruct((NTOK, D), F32),
            jax.ShapeDtypeStruct((NTOK, D // 2), jnp.int32),
            jax.ShapeDtypeStruct((NTOK, 128), F32),
        ],
    )(attn, Wo.astype(BF16), bo.reshape(1, D), xf, g2.reshape(1, D),
      b2.reshape(1, D), Wgp)


def _route_body(lg_ref, d0_ref, d1_ref, g0_ref, g1_ref, w0_ref, w1_ref, cnt_ref):
    lg = lg_ref[0]
    col = jax.lax.broadcasted_iota(jnp.int32, (N, 128), 1)
    valid = col < E
    lgm = jnp.where(valid, lg, -1e30)
    mx = jnp.max(lgm, axis=1, keepdims=True)
    p = jnp.where(valid, jnp.exp(lgm - mx), 0.0)
    probs = p / jnp.sum(p, axis=1, keepdims=True)
    w0 = jnp.max(probs, axis=1, keepdims=True)
    i0 = jnp.min(jnp.where((probs == w0) & valid, col, 128), axis=1, keepdims=True)
    oh0 = col == i0
    probs1 = jnp.where(oh0 | ~valid, -1.0, probs)
    w1 = jnp.max(probs1, axis=1, keepdims=True)
    i1 = jnp.min(jnp.where(probs1 == w1, col, 128), axis=1, keepdims=True)
    oh1 = col == i1
    wsum = w0 + w1
    w0n = w0 / wsum
    w1n = w1 / wsum
    ohs = oh0.astype(F32) + oh1.astype(F32)
    y = ohs
    sh = 1
    while sh < N:
        y = y + jnp.concatenate([jnp.zeros((sh, 128), F32), y[:N - sh]], axis=0)
        sh *= 2
    cnt_ref[0] = y[N - 1:N]
    Sx = y - ohs
    oh0f = oh0.astype(F32)
    r0 = jnp.sum(Sx * oh0f, axis=1, keepdims=True).astype(jnp.int32)
    r1 = jnp.sum((Sx + oh0f) * oh1.astype(F32), axis=1, keepdims=True).astype(jnp.int32)
    keep0 = r0 < C
    keep1 = r1 < C
    d0_ref[0] = jnp.where(keep0, i0 * C + r0, DROP)
    d1_ref[0] = jnp.where(keep1, i1 * C + r1, DROP)
    g0_ref[0] = i0 * C + jnp.minimum(r0, C - 1)
    g1_ref[0] = i1 * C + jnp.minimum(r1, C - 1)
    w0_ref[0] = jnp.where(keep0, w0n, 0.0)
    w1_ref[0] = jnp.where(keep1, w1n, 0.0)


def _route(logits):
    ospec = pl.BlockSpec((1, N, 1), lambda c: (c, 0, 0))
    oshape_i = jax.ShapeDtypeStruct((NCH, N, 1), jnp.int32)
    oshape_f = jax.ShapeDtypeStruct((NCH, N, 1), F32)
    return pl.pallas_call(
        _route_body,
        grid=(NCH,),
        in_specs=[pl.BlockSpec((1, N, 128), lambda c: (c, 0, 0))],
        out_specs=[ospec] * 6 + [pl.BlockSpec((1, 1, 128), lambda c: (c, 0, 0))],
        out_shape=[oshape_i, oshape_i, oshape_i, oshape_i, oshape_f, oshape_f,
                   jax.ShapeDtypeStruct((NCH, 1, 128), F32)],
    )(logits.reshape(NCH, N, 128))


NW = 32
SCB = 128
CPAD = E * C + 8


def _sc_mesh():
    return plsc.VectorSubcoreMesh(core_axis_name="c", subcore_axis_name="s")


def _sc_dispatch(table, srcidx, dstidx):
    nidx = srcidx.shape[0]
    per_w = nidx // NW

    @functools.partial(
        pl.kernel, mesh=_sc_mesh(),
        out_type=jax.ShapeDtypeStruct((NCH * CPAD, D // 2), jnp.int32),
        scratch_types=[
            pltpu.VMEM((SCB,), jnp.int32),
            pltpu.VMEM((SCB,), jnp.int32),
            pltpu.VMEM((SCB, D // 2), jnp.int32),
            pltpu.SemaphoreType.DMA,
            pltpu.SemaphoreType.DMA,
        ],
    )
    def k(table_hbm, src_hbm, dst_hbm, out_hbm, src_v, dst_v, rows_v, s1, s2):
        wid = jax.lax.axis_index("s") * 2 + jax.lax.axis_index("c")

        def body(j, carry):
            base = pl.multiple_of(wid * per_w + j * SCB, SCB)
            pltpu.sync_copy(src_hbm.at[pl.ds(base, SCB)], src_v)
            pltpu.async_copy(table_hbm.at[src_v], rows_v, s1).wait()
            pltpu.sync_copy(dst_hbm.at[pl.ds(base, SCB)], dst_v)
            pltpu.async_copy(rows_v, out_hbm.at[dst_v], s2).wait()
            return carry

        jax.lax.fori_loop(0, per_w // SCB, body, 0)

    return k(table, srcidx, dstidx)


def _sc_gather(table, idx):
    nidx = idx.shape[0]
    per_w = nidx // NW

    @functools.partial(
        pl.kernel, mesh=_sc_mesh(),
        out_type=jax.ShapeDtypeStruct((nidx, D), F32),
        scratch_types=[
            pltpu.VMEM((SCB,), jnp.int32),
            pltpu.VMEM((SCB, D), F32),
            pltpu.SemaphoreType.DMA,
        ],
    )
    def k(table_hbm, idx_hbm, out_hbm, idx_v, rows_v, s1):
        wid = jax.lax.axis_index("s") * 2 + jax.lax.axis_index("c")

        def body(j, carry):
            base = pl.multiple_of(wid * per_w + j * SCB, SCB)
            pltpu.sync_copy(idx_hbm.at[pl.ds(base, SCB)], idx_v)
            pltpu.async_copy(table_hbm.at[idx_v], rows_v, s1).wait()
            pltpu.sync_copy(rows_v, out_hbm.at[pl.ds(base, SCB)])
            return carry

        jax.lax.fori_loop(0, per_w // SCB, body, 0)

    return k(table, idx)


TB = 512


def _wsum_body(w0_ref, w1_ref, x_ref, a_ref, b_ref, o_ref):
    o_ref[...] = (x_ref[...] + w0_ref[0] * a_ref[...] + w1_ref[0] * b_ref[...])


def _wsum(w0, w1, xout, gath):
    nt = N // TB
    return pl.pallas_call(
        _wsum_body,
        grid=(NCH, nt),
        in_specs=[
            pl.BlockSpec((1, TB, 1), lambda c, t: (c, t, 0)),
            pl.BlockSpec((1, TB, 1), lambda c, t: (c, t, 0)),
            pl.BlockSpec((TB, D), lambda c, t: (c * (N // TB) + t, 0)),
            pl.BlockSpec((TB, D), lambda c, t: (c * (N // TB) + t, 0)),
            pl.BlockSpec((TB, D), lambda c, t: (NTOK // TB + c * (N // TB) + t, 0)),
        ],
        out_specs=pl.BlockSpec((TB, D), lambda c, t: (c * (N // TB) + t, 0)),
        out_shape=jax.ShapeDtypeStruct((NTOK, D), F32),
    )(w0, w1, xout, gath, gath)


CB = 512


def _ffn_body(cnt_ref, buf_ref, w1_ref, b1_ref, w2_ref, b2_ref, o_ref, w1s, w2s):
    e = pl.program_id(0)
    c = pl.program_id(1)
    rb = pl.program_id(2)

    @pl.when((c == 0) & (rb == 0))
    def _():
        w1s[...] = w1_ref[0].astype(BF16)
        w2s[...] = w2_ref[0].astype(BF16)

    @pl.when(rb * CB < cnt_ref[c * E + e])
    def _():
        w = buf_ref[0]
        lo = jax.lax.bitcast_convert_type(jax.lax.shift_left(w, 16), F32)
        hi = jax.lax.bitcast_convert_type(w & jnp.int32(-65536), F32)
        x = jnp.concatenate([lo, hi], axis=1).astype(BF16)
        h = jnp.dot(x, w1s[...], preferred_element_type=F32) + b1_ref[0]
        h = jax.nn.gelu(h)
        o_ref[0] = jnp.dot(h.astype(BF16), w2s[...],
                           preferred_element_type=F32) + b2_ref[0]


def _ffn(cnt, buf, W1, b1r, W2, b2r):
    grid_spec = pltpu.PrefetchScalarGridSpec(
        num_scalar_prefetch=1,
        grid=(E, NCH, C // CB),
        in_specs=[
            pl.BlockSpec((1, CB, D // 2),
                         lambda e, c, r, s: (c, e * (C // CB) + r, 0)),
            pl.BlockSpec((1, D, DFF), lambda e, c, r, s: (e, 0, 0)),
            pl.BlockSpec((1, 1, DFF), lambda e, c, r, s: (e, 0, 0)),
            pl.BlockSpec((1, DFF, D), lambda e, c, r, s: (e, 0, 0)),
            pl.BlockSpec((1, 1, D), lambda e, c, r, s: (e, 0, 0)),
        ],
        out_specs=pl.BlockSpec((1, CB, D),
                               lambda e, c, r, s: (c, e * (C // CB) + r, 0)),
        scratch_shapes=[
            pltpu.VMEM((D, DFF), BF16),
            pltpu.VMEM((DFF, D), BF16),
        ],
    )
    return pl.pallas_call(
        _ffn_body,
        grid_spec=grid_spec,
        out_shape=jax.ShapeDtypeStruct((NCH, E * C, D), F32),
    )(cnt, buf, W1, b1r, W2, b2r)


def kernel(x, gamma1, beta1, Wqkv, bqkv, Wo, bo, gamma2, beta2, Wg, W1, b1, W2, b2):
    xf = x.reshape(NTOK, D)
    qkv = _qkv(xf, gamma1, beta1, Wqkv, bqkv)
    attn = _attn(qkv)
    Wgp = jnp.pad(Wg, ((0, 0), (0, 128 - E)))
    xout, lnf, logits = _proj(attn, Wo, bo, xf, gamma2, beta2, Wgp)
    d0, d1, g0, g1, w0, w1, cnte = _route(logits)
    cnt = cnte[:, 0, :E].astype(jnp.int32).reshape(NCH * E)
    coff = (jnp.arange(NCH, dtype=jnp.int32) * CPAD)[:, None, None]
    dst = jnp.concatenate([(d0 + coff).reshape(-1), (d1 + coff).reshape(-1)])
    src = jnp.concatenate([jnp.arange(NTOK, dtype=jnp.int32)] * 2)
    buf = _sc_dispatch(lnf, src, dst).reshape(NCH, CPAD, D // 2)
    eo = _ffn(cnt, buf, W1, b1.reshape(E, 1, DFF),
              W2, b2.reshape(E, 1, D))
    goff = (jnp.arange(NCH, dtype=jnp.int32) * (E * C))[:, None, None]
    gcat = jnp.concatenate([(g0 + goff).reshape(-1), (g1 + goff).reshape(-1)])
    gath = _sc_gather(eo.reshape(NCH * E * C, D), gcat)
    out = _wsum(w0, w1, xout, gath)
    return out.reshape(B, T, D)
